# baseline (speedup 1.0000x reference)
import functools

import jax
import jax.numpy as jnp
from jax import lax
from jax.experimental import pallas as pl
from jax.experimental.pallas import tpu as pltpu
from jax.experimental.pallas import tpu_sc as plsc

N = 10000
E = 320000
D = 128

NC = 2
NS = 16
LANES = 16

EPW = E // (NC * NS)
CHUNK = 64
NCHUNK = EPW // CHUNK
NBUF = 4
TAIL = EPW - NCHUNK * CHUNK
NPAD = 10240
ROWS_PW = NPAD // NS
ZROWS = 16


def _segsum_body(vals, srch, dsth, out,
                 dst_all, sv0, sv1, sv2, sv3, dv0, dv1, dv2, dv3, dvt,
                 rows0, rows1, rows2, rows3, zb, agg,
                 dsem, isem0, isem1, isem2, isem3,
                 gsem0, gsem1, gsem2, gsem3, ssem0, ssem1, ssem2, ssem3):
    c = lax.axis_index("c")
    s = lax.axis_index("s")
    rows = (rows0, rows1, rows2, rows3)
    sv = (sv0, sv1, sv2, sv3)
    dv = (dv0, dv1, dv2, dv3)
    isem = (isem0, isem1, isem2, isem3)
    gsem = (gsem0, gsem1, gsem2, gsem3)
    ssem = (ssem0, ssem1, ssem2, ssem3)

    base = (c * NS + s) * EPW

    cp_dst = pltpu.async_copy(dsth.at[pl.ds(base, EPW)], dst_all, dsem)
    pltpu.async_copy(srch.at[pl.ds(base, CHUNK)], sv0, isem0)
    pltpu.async_copy(srch.at[pl.ds(base + CHUNK, CHUNK)], sv1, isem1)

    zvec = jnp.zeros((LANES,), jnp.float32)

    def _zb_loop(t, _):
        i = t // (D // LANES)
        j = t % (D // LANES)
        zb[i, pl.ds(j * LANES, LANES)] = zvec
        return 0

    lax.fori_loop(0, ZROWS * (D // LANES), _zb_loop, 0)

    def _zero_loop(j, _):
        pltpu.sync_copy(zb, agg.at[pl.ds(s * ROWS_PW + j * ZROWS, ZROWS)])
        return 0

    lax.fori_loop(0, ROWS_PW // ZROWS, _zero_loop, 0)

    cp_dst.wait()
    plsc.subcore_barrier()

    def idx_copy(chunk_start, dvb, n):
        for j in range(n // LANES):
            dvb[pl.ds(j * LANES, LANES)] = (
                dst_all[pl.ds(chunk_start + j * LANES, LANES)])

    def issue_isrc(chunk, b):
        pltpu.async_copy(srch.at[pl.ds(base + chunk * CHUNK, CHUNK)],
                         sv[b], isem[b])

    def wait_isrc(b):
        pltpu.make_async_copy(srch.at[pl.ds(0, CHUNK)], sv[b],
                              isem[b]).wait()

    def issue_gather(b):
        pltpu.async_copy(vals.at[sv[b]], rows[b], gsem[b])

    def issue_scatter(b):
        pltpu.async_copy(rows[b], agg.at[dv[b]], ssem[b], add=True)

    def wait_g(b):
        pltpu.make_async_copy(vals.at[pl.ds(0, CHUNK)], rows[b],
                              gsem[b]).wait()

    def wait_s(b):
        pltpu.make_async_copy(vals.at[pl.ds(0, CHUNK)], rows[b],
                              ssem[b]).wait()

    idx_copy(0, dv[0], CHUNK)
    wait_isrc(0)
    issue_gather(0)

    def _pipe_body(i2, _):
        for b in range(NBUF):
            i = NBUF * i2 + b
            nb = (b + 1) % NBUF
            wait_g(b)
            issue_scatter(b)

            @pl.when(i + 2 < NCHUNK)
            def _():
                issue_isrc(i + 2, (b + 2) % NBUF)

            @pl.when(i >= NBUF - 1)
            def _():
                wait_s(nb)

            @pl.when(i + 1 < NCHUNK)
            def _():
                idx_copy((i + 1) * CHUNK, dv[nb], CHUNK)
                wait_isrc(nb)
                issue_gather(nb)
        return 0

    lax.fori_loop(0, NCHUNK // NBUF, _pipe_body, 0)
    for j in range(NCHUNK - NBUF + 1, NCHUNK):
        wait_s(j % NBUF)

    idx_copy(NCHUNK * CHUNK, dvt, TAIL)
    pltpu.async_copy(srch.at[pl.ds(base + NCHUNK * CHUNK, TAIL)],
                     sv0.at[pl.ds(0, TAIL)], isem0)
    pltpu.make_async_copy(srch.at[pl.ds(0, TAIL)], sv0.at[pl.ds(0, TAIL)],
                          isem0).wait()
    pltpu.async_copy(vals.at[sv0.at[pl.ds(0, TAIL)]],
                     rows0.at[pl.ds(0, TAIL)], gsem0).wait()
    pltpu.sync_copy(rows0.at[pl.ds(0, TAIL)], agg.at[dvt], add=True)

    plsc.subcore_barrier()

    pltpu.sync_copy(
        agg.at[pl.ds(s * ROWS_PW, ROWS_PW)],
        out.at[c].at[pl.ds(s * ROWS_PW, ROWS_PW)],
    )


def _segsum_sc(vals, src, dst):
    mesh = plsc.VectorSubcoreMesh(
        core_axis_name="c", subcore_axis_name="s",
        num_cores=NC, num_subcores=NS,
    )
    f = pl.kernel(
        _segsum_body,
        out_type=jax.ShapeDtypeStruct((NC, NPAD, D), jnp.float32),
        mesh=mesh,
        scratch_types=(
            [pltpu.VMEM((EPW,), jnp.int32)]
            + [pltpu.VMEM((CHUNK,), jnp.int32)] * NBUF
            + [pltpu.VMEM((CHUNK,), jnp.int32)] * NBUF
            + [pltpu.VMEM((TAIL,), jnp.int32)]
            + [pltpu.VMEM((CHUNK, D), jnp.float32)] * NBUF
            + [pltpu.VMEM((ZROWS, D), jnp.float32)]
            + [pltpu.VMEM_SHARED((NPAD, D), jnp.float32)]
            + [pltpu.SemaphoreType.DMA] * (1 + 3 * NBUF)
        ),
    )
    return f(vals, src, dst)


def _dense_body(relu, a0, a1, xr, wrel, wroot, br, o):
    acc = jnp.dot(a0[...] + a1[...], wrel[...],
                  preferred_element_type=jnp.float32)
    acc += jnp.dot(xr[...], wroot[...], preferred_element_type=jnp.float32)
    acc += br[...]
    o[...] = jnp.maximum(acc, 0.0) if relu else acc


def _dense_tc(aggp, xin, wrel, b, wroot, relu):
    bm = 1000
    grid = (N // bm,)
    row_spec = pl.BlockSpec((bm, D), lambda i: (i, 0))
    w_spec = pl.BlockSpec((D, D), lambda i: (0, 0))
    return pl.pallas_call(
        functools.partial(_dense_body, relu),
        grid=grid,
        in_specs=[row_spec, row_spec, row_spec, w_spec, w_spec,
                  pl.BlockSpec((1, D), lambda i: (0, 0))],
        out_specs=row_spec,
        out_shape=jax.ShapeDtypeStruct((N, D), jnp.float32),
    )(aggp[0], aggp[1], xin, wrel, wroot, b.reshape(1, D))


def kernel(x, edge_index, W_rel0, b_rel0, W_root0, W_rel1, b_rel1, W_root1):
    src = edge_index[0]
    dst = edge_index[1]
    agg0 = _segsum_sc(x, src, dst)
    h = _dense_tc(agg0, x, W_rel0, b_rel0, W_root0, relu=True)
    agg1 = _segsum_sc(h, src, dst)
    out = _dense_tc(agg1, h, W_rel1, b_rel1, W_root1, relu=False)
    return out


# speedup vs baseline: 1.2842x; 1.2842x over previous
import functools

import jax
import jax.numpy as jnp
from jax import lax
from jax.experimental import pallas as pl
from jax.experimental.pallas import tpu as pltpu
from jax.experimental.pallas import tpu_sc as plsc

N = 10000
E = 320000
D = 128

NC = 2
NS = 16
LANES = 16

EPW = E // (NC * NS)
CHUNK = 128
NCHUNK = EPW // CHUNK
NBUF = 2
TAIL = EPW - NCHUNK * CHUNK
NPAD = 10240
ROWS_PW = NPAD // NS
ZROWS = 16


def _segsum_body(vals, srch, dsth, out,
                 dst_all, sv0, sv1, dv0, dv1, dvt,
                 rows0, rows1, zb, agg,
                 dsem, isem0, isem1, gsem0, gsem1, ssem0, ssem1):
    c = lax.axis_index("c")
    s = lax.axis_index("s")
    rows = (rows0, rows1)
    sv = (sv0, sv1)
    dv = (dv0, dv1)
    isem = (isem0, isem1)
    gsem = (gsem0, gsem1)
    ssem = (ssem0, ssem1)

    base = (c * NS + s) * EPW

    cp_dst = pltpu.async_copy(dsth.at[pl.ds(base, EPW)], dst_all, dsem)
    pltpu.async_copy(srch.at[pl.ds(base, CHUNK)], sv0, isem0)
    pltpu.async_copy(srch.at[pl.ds(base + CHUNK, CHUNK)], sv1, isem1)

    zvec = jnp.zeros((LANES,), jnp.float32)

    def _zb_loop(t, _):
        i = t // (D // LANES)
        j = t % (D // LANES)
        zb[i, pl.ds(j * LANES, LANES)] = zvec
        return 0

    lax.fori_loop(0, ZROWS * (D // LANES), _zb_loop, 0)

    def _zero_loop(j, _):
        pltpu.sync_copy(zb, agg.at[pl.ds(s * ROWS_PW + j * ZROWS, ZROWS)])
        return 0

    lax.fori_loop(0, ROWS_PW // ZROWS, _zero_loop, 0)

    cp_dst.wait()
    plsc.subcore_barrier()

    def idx_copy(chunk_start, dvb, n):
        for j in range(n // LANES):
            dvb[pl.ds(j * LANES, LANES)] = (
                dst_all[pl.ds(chunk_start + j * LANES, LANES)])

    def issue_isrc(chunk, b):
        pltpu.async_copy(srch.at[pl.ds(base + chunk * CHUNK, CHUNK)],
                         sv[b], isem[b])

    def wait_isrc(b):
        pltpu.make_async_copy(srch.at[pl.ds(0, CHUNK)], sv[b],
                              isem[b]).wait()

    def issue_gather(b):
        pltpu.async_copy(vals.at[sv[b]], rows[b], gsem[b])

    def issue_scatter(b):
        pltpu.async_copy(rows[b], agg.at[dv[b]], ssem[b], add=True)

    def wait_g(b):
        pltpu.make_async_copy(vals.at[pl.ds(0, CHUNK)], rows[b],
                              gsem[b]).wait()

    def wait_s(b):
        pltpu.make_async_copy(vals.at[pl.ds(0, CHUNK)], rows[b],
                              ssem[b]).wait()

    idx_copy(0, dv[0], CHUNK)
    wait_isrc(0)
    issue_gather(0)

    def _pipe_body(i2, _):
        for b in range(NBUF):
            i = NBUF * i2 + b
            nb = (b + 1) % NBUF
            wait_g(b)
            issue_scatter(b)

            @pl.when(i + 2 < NCHUNK)
            def _():
                issue_isrc(i + 2, (b + 2) % NBUF)

            @pl.when(i >= NBUF - 1)
            def _():
                wait_s(nb)

            @pl.when(i + 1 < NCHUNK)
            def _():
                idx_copy((i + 1) * CHUNK, dv[nb], CHUNK)
                wait_isrc(nb)
                issue_gather(nb)
        return 0

    lax.fori_loop(0, NCHUNK // NBUF, _pipe_body, 0)
    for j in range(NCHUNK - NBUF + 1, NCHUNK):
        wait_s(j % NBUF)

    idx_copy(NCHUNK * CHUNK, dvt, TAIL)
    pltpu.async_copy(srch.at[pl.ds(base + NCHUNK * CHUNK, TAIL)],
                     sv0.at[pl.ds(0, TAIL)], isem0)
    pltpu.make_async_copy(srch.at[pl.ds(0, TAIL)], sv0.at[pl.ds(0, TAIL)],
                          isem0).wait()
    pltpu.async_copy(vals.at[sv0.at[pl.ds(0, TAIL)]],
                     rows0.at[pl.ds(0, TAIL)], gsem0).wait()
    pltpu.sync_copy(rows0.at[pl.ds(0, TAIL)], agg.at[dvt], add=True)

    plsc.subcore_barrier()

    pltpu.sync_copy(
        agg.at[pl.ds(s * ROWS_PW, ROWS_PW)],
        out.at[c].at[pl.ds(s * ROWS_PW, ROWS_PW)],
    )


def _segsum_sc(vals, src, dst):
    mesh = plsc.VectorSubcoreMesh(
        core_axis_name="c", subcore_axis_name="s",
        num_cores=NC, num_subcores=NS,
    )
    f = pl.kernel(
        _segsum_body,
        out_type=jax.ShapeDtypeStruct((NC, NPAD, D), jnp.float32),
        mesh=mesh,
        scratch_types=(
            [pltpu.VMEM((EPW,), jnp.int32)]
            + [pltpu.VMEM((CHUNK,), jnp.int32)] * NBUF
            + [pltpu.VMEM((CHUNK,), jnp.int32)] * NBUF
            + [pltpu.VMEM((TAIL,), jnp.int32)]
            + [pltpu.VMEM((CHUNK, D), jnp.float32)] * NBUF
            + [pltpu.VMEM((ZROWS, D), jnp.float32)]
            + [pltpu.VMEM_SHARED((NPAD, D), jnp.float32)]
            + [pltpu.SemaphoreType.DMA] * (1 + 3 * NBUF)
        ),
    )
    return f(vals, src, dst)


_BM = 1000
_ROW_SPEC = pl.BlockSpec((_BM, D), lambda i: (i, 0))
_W_SPEC = pl.BlockSpec((D, D), lambda i: (0, 0))


def _root_body(xr, wroot, br, o):
    o[...] = jnp.dot(xr[...], wroot[...],
                     preferred_element_type=jnp.float32) + br[...]


def _root_tc(xin, wroot, b):
    return pl.pallas_call(
        _root_body,
        grid=(N // _BM,),
        in_specs=[_ROW_SPEC, _W_SPEC, pl.BlockSpec((1, D), lambda i: (0, 0))],
        out_specs=_ROW_SPEC,
        out_shape=jax.ShapeDtypeStruct((N, D), jnp.float32),
    )(xin, wroot, b.reshape(1, D))


def _combine_body(relu, a0, a1, root, wrel, o):
    acc = jnp.dot(a0[...] + a1[...], wrel[...],
                  preferred_element_type=jnp.float32)
    acc += root[...]
    o[...] = jnp.maximum(acc, 0.0) if relu else acc


def _combine_tc(aggp, root, wrel, relu):
    return pl.pallas_call(
        functools.partial(_combine_body, relu),
        grid=(N // _BM,),
        in_specs=[_ROW_SPEC, _ROW_SPEC, _ROW_SPEC, _W_SPEC],
        out_specs=_ROW_SPEC,
        out_shape=jax.ShapeDtypeStruct((N, D), jnp.float32),
    )(aggp[0], aggp[1], root, wrel)


def kernel(x, edge_index, W_rel0, b_rel0, W_root0, W_rel1, b_rel1, W_root1):
    src = edge_index[0]
    dst = edge_index[1]
    root0 = _root_tc(x, W_root0, b_rel0)
    agg0 = _segsum_sc(x, src, dst)
    h = _combine_tc(agg0, root0, W_rel0, relu=True)
    root1 = _root_tc(h, W_root1, b_rel1)
    agg1 = _segsum_sc(h, src, dst)
    out = _combine_tc(agg1, root1, W_rel1, relu=False)
    return out


# speedup vs baseline: 1.3916x; 1.0836x over previous
import functools

import jax
import jax.numpy as jnp
from jax import lax
from jax.experimental import pallas as pl
from jax.experimental.pallas import tpu as pltpu
from jax.experimental.pallas import tpu_sc as plsc

N = 10000
E = 320000
D = 128

NC = 2
NS = 16
LANES = 16

EPW = E // (NC * NS)
CHUNK = 128
NCHUNK = EPW // CHUNK
NBUF = 2
TAIL = EPW - NCHUNK * CHUNK
NPAD = 10240
ROWS_PW = NPAD // NS
ZROWS = 16


def _segsum_body(vals, eidx, out,
                 dst_all, sv0, sv1, dv0, dv1, dvt,
                 rows0, rows1, zb, agg,
                 dsem, isem0, isem1, gsem0, gsem1, ssem0, ssem1):
    c = lax.axis_index("c")
    s = lax.axis_index("s")
    rows = (rows0, rows1)
    sv = (sv0, sv1)
    dv = (dv0, dv1)
    isem = (isem0, isem1)
    gsem = (gsem0, gsem1)
    ssem = (ssem0, ssem1)

    base = (c * NS + s) * EPW

    cp_dst = pltpu.async_copy(eidx.at[pl.ds(E + base, EPW)], dst_all, dsem)
    pltpu.async_copy(eidx.at[pl.ds(base, CHUNK)], sv0, isem0)
    pltpu.async_copy(eidx.at[pl.ds(base + CHUNK, CHUNK)], sv1, isem1)

    zvec = jnp.zeros((LANES,), jnp.float32)

    def _zb_loop(t, _):
        i = t // (D // LANES)
        j = t % (D // LANES)
        zb[i, pl.ds(j * LANES, LANES)] = zvec
        return 0

    lax.fori_loop(0, ZROWS * (D // LANES), _zb_loop, 0)

    def _zero_loop(j, _):
        pltpu.sync_copy(zb, agg.at[pl.ds(s * ROWS_PW + j * ZROWS, ZROWS)])
        return 0

    lax.fori_loop(0, ROWS_PW // ZROWS, _zero_loop, 0)

    cp_dst.wait()
    plsc.subcore_barrier()

    def idx_copy(chunk_start, dvb, n):
        for j in range(n // LANES):
            dvb[pl.ds(j * LANES, LANES)] = (
                dst_all[pl.ds(chunk_start + j * LANES, LANES)])

    def issue_isrc(chunk, b):
        pltpu.async_copy(eidx.at[pl.ds(base + chunk * CHUNK, CHUNK)],
                         sv[b], isem[b])

    def wait_isrc(b):
        pltpu.make_async_copy(eidx.at[pl.ds(0, CHUNK)], sv[b],
                              isem[b]).wait()

    def issue_gather(b):
        pltpu.async_copy(vals.at[sv[b]], rows[b], gsem[b])

    def issue_scatter(b):
        pltpu.async_copy(rows[b], agg.at[dv[b]], ssem[b], add=True)

    def wait_g(b):
        pltpu.make_async_copy(vals.at[pl.ds(0, CHUNK)], rows[b],
                              gsem[b]).wait()

    def wait_s(b):
        pltpu.make_async_copy(vals.at[pl.ds(0, CHUNK)], rows[b],
                              ssem[b]).wait()

    idx_copy(0, dv[0], CHUNK)
    wait_isrc(0)
    issue_gather(0)

    def _pipe_body(i2, _):
        for b in range(NBUF):
            i = NBUF * i2 + b
            nb = (b + 1) % NBUF
            wait_g(b)
            issue_scatter(b)

            @pl.when(i + 2 < NCHUNK)
            def _():
                issue_isrc(i + 2, (b + 2) % NBUF)

            @pl.when(i >= NBUF - 1)
            def _():
                wait_s(nb)

            @pl.when(i + 1 < NCHUNK)
            def _():
                idx_copy((i + 1) * CHUNK, dv[nb], CHUNK)
                wait_isrc(nb)
                issue_gather(nb)
        return 0

    lax.fori_loop(0, NCHUNK // NBUF, _pipe_body, 0)
    for j in range(NCHUNK - NBUF + 1, NCHUNK):
        wait_s(j % NBUF)

    idx_copy(NCHUNK * CHUNK, dvt, TAIL)
    pltpu.async_copy(eidx.at[pl.ds(base + NCHUNK * CHUNK, TAIL)],
                     sv0.at[pl.ds(0, TAIL)], isem0)
    pltpu.make_async_copy(eidx.at[pl.ds(0, TAIL)], sv0.at[pl.ds(0, TAIL)],
                          isem0).wait()
    pltpu.async_copy(vals.at[sv0.at[pl.ds(0, TAIL)]],
                     rows0.at[pl.ds(0, TAIL)], gsem0).wait()
    pltpu.sync_copy(rows0.at[pl.ds(0, TAIL)], agg.at[dvt], add=True)

    plsc.subcore_barrier()

    pltpu.sync_copy(
        agg.at[pl.ds(s * ROWS_PW, ROWS_PW)],
        out.at[c].at[pl.ds(s * ROWS_PW, ROWS_PW)],
    )


def _segsum_sc(vals, edge_index):
    mesh = plsc.VectorSubcoreMesh(
        core_axis_name="c", subcore_axis_name="s",
        num_cores=NC, num_subcores=NS,
    )
    f = pl.kernel(
        _segsum_body,
        out_type=jax.ShapeDtypeStruct((NC, NPAD, D), jnp.float32),
        mesh=mesh,
        scratch_types=(
            [pltpu.VMEM((EPW,), jnp.int32)]
            + [pltpu.VMEM((CHUNK,), jnp.int32)] * NBUF
            + [pltpu.VMEM((CHUNK,), jnp.int32)] * NBUF
            + [pltpu.VMEM((TAIL,), jnp.int32)]
            + [pltpu.VMEM((CHUNK, D), jnp.float32)] * NBUF
            + [pltpu.VMEM((ZROWS, D), jnp.float32)]
            + [pltpu.VMEM_SHARED((NPAD, D), jnp.float32)]
            + [pltpu.SemaphoreType.DMA] * (1 + 3 * NBUF)
        ),
    )
    return f(vals, edge_index.reshape(2 * E))


_BM = 1000
_ROW_SPEC = pl.BlockSpec((_BM, D), lambda i: (i, 0))
_W_SPEC = pl.BlockSpec((D, D), lambda i: (0, 0))


def _root_body(xr, wroot, br, o):
    o[...] = jnp.dot(xr[...], wroot[...],
                     preferred_element_type=jnp.float32) + br[...]


def _root_tc(xin, wroot, b):
    return pl.pallas_call(
        _root_body,
        grid=(N // _BM,),
        in_specs=[_ROW_SPEC, _W_SPEC, pl.BlockSpec((1, D), lambda i: (0, 0))],
        out_specs=_ROW_SPEC,
        out_shape=jax.ShapeDtypeStruct((N, D), jnp.float32),
    )(xin, wroot, b.reshape(1, D))


def _combine_body(relu, a0, a1, root, wrel, o):
    acc = jnp.dot(a0[0] + a1[0], wrel[...],
                  preferred_element_type=jnp.float32)
    acc += root[...]
    o[...] = jnp.maximum(acc, 0.0) if relu else acc


def _combine_tc(aggp, root, wrel, relu):
    return pl.pallas_call(
        functools.partial(_combine_body, relu),
        grid=(N // _BM,),
        in_specs=[pl.BlockSpec((1, _BM, D), lambda i: (0, i, 0)),
                  pl.BlockSpec((1, _BM, D), lambda i: (1, i, 0)),
                  _ROW_SPEC, _W_SPEC],
        out_specs=_ROW_SPEC,
        out_shape=jax.ShapeDtypeStruct((N, D), jnp.float32),
    )(aggp, aggp, root, wrel)


def kernel(x, edge_index, W_rel0, b_rel0, W_root0, W_rel1, b_rel1, W_root1):
    root0 = _root_tc(x, W_root0, b_rel0)
    agg0 = _segsum_sc(x, edge_index)
    h = _combine_tc(agg0, root0, W_rel0, relu=True)
    root1 = _root_tc(h, W_root1, b_rel1)
    agg1 = _segsum_sc(h, edge_index)
    out = _combine_tc(agg1, root1, W_rel1, relu=False)
    return out


# speedup vs baseline: 1.6051x; 1.1534x over previous
import functools

import jax
import jax.numpy as jnp
from jax import lax
from jax.experimental import pallas as pl
from jax.experimental.pallas import tpu as pltpu
from jax.experimental.pallas import tpu_sc as plsc

N = 10000
E = 320000
D = 128

NC = 2
NS = 16
LANES = 16

EPW = E // (NC * NS)
CHUNK = 128
NCHUNK = EPW // CHUNK
TAIL = EPW - NCHUNK * CHUNK
NR = 3
NSV = 4
NDV = 2
UNROLL = 12
NMAIN = (NCHUNK // UNROLL) * UNROLL
NPAD = 10112
ROWS_PW = NPAD // NS
ZROWS = 32


def _segsum_body(vals, eidx, out,
                 sv0, sv1, sv2, sv3, dv0, dv1, dvt,
                 rows0, rows1, rows2, agg,
                 isem0, isem1, isem2, isem3, dsem0, dsem1,
                 gsem0, gsem1, gsem2):
    c = lax.axis_index("c")
    s = lax.axis_index("s")
    rows = (rows0, rows1, rows2)
    sv = (sv0, sv1, sv2, sv3)
    dv = (dv0, dv1)
    isem = (isem0, isem1, isem2, isem3)
    dsem = (dsem0, dsem1)
    gsem = (gsem0, gsem1, gsem2)

    base = (c * NS + s) * EPW

    def issue_isrc(chunk, slot):
        pltpu.async_copy(eidx.at[pl.ds(base + chunk * CHUNK, CHUNK)],
                         sv[slot], isem[slot])

    def wait_isrc(slot):
        pltpu.make_async_copy(eidx.at[pl.ds(0, CHUNK)], sv[slot],
                              isem[slot]).wait()

    def issue_idst(chunk, slot):
        pltpu.async_copy(eidx.at[pl.ds(E + base + chunk * CHUNK, CHUNK)],
                         dv[slot], dsem[slot])

    def wait_idst(slot):
        pltpu.make_async_copy(eidx.at[pl.ds(0, CHUNK)], dv[slot],
                              dsem[slot]).wait()

    def issue_gather(sslot, rslot):
        pltpu.async_copy(vals.at[sv[sslot]], rows[rslot], gsem[rslot])

    def wait_g(rslot):
        pltpu.make_async_copy(vals.at[pl.ds(0, CHUNK)], rows[rslot],
                              gsem[rslot]).wait()

    def scatter_sync(rslot, dslot):
        pltpu.sync_copy(rows[rslot], agg.at[dv[dslot]], add=True)

    for k in range(NSV):
        issue_isrc(k, k)
    issue_idst(0, 0)

    zvec = jnp.zeros((LANES,), jnp.float32)

    def _zb_loop(t, _):
        rows0[t // (D // LANES), pl.ds((t % (D // LANES)) * LANES, LANES)] = (
            zvec)
        return 0

    lax.fori_loop(0, ZROWS * (D // LANES), _zb_loop, 0)

    def _zero_loop(j, _):
        pltpu.sync_copy(rows0.at[pl.ds(0, ZROWS)],
                        agg.at[pl.ds(s * ROWS_PW + j * ZROWS, ZROWS)])
        return 0

    lax.fori_loop(0, (ROWS_PW // ZROWS), _zero_loop, 0)
    pltpu.sync_copy(rows0.at[pl.ds(0, ROWS_PW % ZROWS)],
                    agg.at[pl.ds(s * ROWS_PW + (ROWS_PW // ZROWS) * ZROWS,
                                 ROWS_PW % ZROWS)])

    plsc.subcore_barrier()

    wait_isrc(0)
    issue_gather(0, 0)
    wait_isrc(1)
    issue_gather(1, 1)

    def _step(i, u, g4=True, g2=True, g1=True):
        wait_g(u % NR)
        if g4:
            issue_isrc(i + 4, u % NSV)
        if g2:
            wait_isrc((u + 2) % NSV)
            issue_gather((u + 2) % NSV, (u + 2) % NR)
        wait_idst(u % NDV)
        scatter_sync(u % NR, u % NDV)
        if g1:
            issue_idst(i + 1, (u + 1) % NDV)

    def _pipe_body(i2, _):
        for u in range(UNROLL):
            _step(UNROLL * i2 + u, u)
        return 0

    lax.fori_loop(0, NMAIN // UNROLL, _pipe_body, 0)

    for i in range(NMAIN, NCHUNK):
        _step(i, i, g4=i + 4 < NCHUNK, g2=i + 2 < NCHUNK,
              g1=i + 1 < NCHUNK)

    pltpu.async_copy(eidx.at[pl.ds(E + base + NCHUNK * CHUNK, TAIL)],
                     dvt, dsem0)
    pltpu.async_copy(eidx.at[pl.ds(base + NCHUNK * CHUNK, TAIL)],
                     sv0.at[pl.ds(0, TAIL)], isem0)
    pltpu.make_async_copy(eidx.at[pl.ds(0, TAIL)], dvt, dsem0).wait()
    pltpu.make_async_copy(eidx.at[pl.ds(0, TAIL)], sv0.at[pl.ds(0, TAIL)],
                          isem0).wait()
    pltpu.async_copy(vals.at[sv0.at[pl.ds(0, TAIL)]],
                     rows0.at[pl.ds(0, TAIL)], gsem0).wait()
    pltpu.sync_copy(rows0.at[pl.ds(0, TAIL)], agg.at[dvt], add=True)

    plsc.subcore_barrier()

    pltpu.sync_copy(
        agg.at[pl.ds(s * ROWS_PW, ROWS_PW)],
        out.at[c].at[pl.ds(s * ROWS_PW, ROWS_PW)],
    )


def _segsum_sc(vals, edge_index):
    mesh = plsc.VectorSubcoreMesh(
        core_axis_name="c", subcore_axis_name="s",
        num_cores=NC, num_subcores=NS,
    )
    f = pl.kernel(
        _segsum_body,
        out_type=jax.ShapeDtypeStruct((NC, NPAD, D), jnp.float32),
        mesh=mesh,
        scratch_types=(
            [pltpu.VMEM((CHUNK,), jnp.int32)] * NSV
            + [pltpu.VMEM((CHUNK,), jnp.int32)] * NDV
            + [pltpu.VMEM((TAIL,), jnp.int32)]
            + [pltpu.VMEM((CHUNK, D), jnp.float32)] * NR
            + [pltpu.VMEM_SHARED((NPAD, D), jnp.float32)]
            + [pltpu.SemaphoreType.DMA] * (NSV + NDV + NR)
        ),
    )
    return f(vals, edge_index.reshape(2 * E))


_BM = 1000
_ROW_SPEC = pl.BlockSpec((_BM, D), lambda i: (i, 0))
_W_SPEC = pl.BlockSpec((D, D), lambda i: (0, 0))


def _root_body(xr, wroot, br, o):
    o[...] = jnp.dot(xr[...], wroot[...],
                     preferred_element_type=jnp.float32) + br[...]


def _root_tc(xin, wroot, b):
    return pl.pallas_call(
        _root_body,
        grid=(N // _BM,),
        in_specs=[_ROW_SPEC, _W_SPEC, pl.BlockSpec((1, D), lambda i: (0, 0))],
        out_specs=_ROW_SPEC,
        out_shape=jax.ShapeDtypeStruct((N, D), jnp.float32),
    )(xin, wroot, b.reshape(1, D))


def _combine_body(relu, a0, a1, root, wrel, o):
    acc = jnp.dot(a0[0] + a1[0], wrel[...],
                  preferred_element_type=jnp.float32)
    acc += root[...]
    o[...] = jnp.maximum(acc, 0.0) if relu else acc


def _combine_tc(aggp, root, wrel, relu):
    return pl.pallas_call(
        functools.partial(_combine_body, relu),
        grid=(N // _BM,),
        in_specs=[pl.BlockSpec((1, _BM, D), lambda i: (0, i, 0)),
                  pl.BlockSpec((1, _BM, D), lambda i: (1, i, 0)),
                  _ROW_SPEC, _W_SPEC],
        out_specs=_ROW_SPEC,
        out_shape=jax.ShapeDtypeStruct((N, D), jnp.float32),
    )(aggp, aggp, root, wrel)


def kernel(x, edge_index, W_rel0, b_rel0, W_root0, W_rel1, b_rel1, W_root1):
    root0 = _root_tc(x, W_root0, b_rel0)
    agg0 = _segsum_sc(x, edge_index)
    h = _combine_tc(agg0, root0, W_rel0, relu=True)
    root1 = _root_tc(h, W_root1, b_rel1)
    agg1 = _segsum_sc(h, edge_index)
    out = _combine_tc(agg1, root1, W_rel1, relu=False)
    return out


# speedup vs baseline: 1.6492x; 1.0274x over previous
import functools

import jax
import jax.numpy as jnp
from jax import lax
from jax.experimental import pallas as pl
from jax.experimental.pallas import tpu as pltpu
from jax.experimental.pallas import tpu_sc as plsc

N = 10000
E = 320000
D = 128

NC = 2
NS = 16
LANES = 16

EPW = E // (NC * NS)
CHUNK = 128
NCHUNK = EPW // CHUNK
TAIL = EPW - NCHUNK * CHUNK
NR = 3
NSV = 4
NDV = 2
UNROLL = 12
NMAIN = (NCHUNK // UNROLL) * UNROLL
NPAD = 10112
ROWS_PW = NPAD // NS
ZROWS = 32


def _segsum_body(vals, eidx, out,
                 sv0, sv1, sv2, sv3, dv0, dv1, dvt,
                 rows0, rows1, rows2, agg,
                 isem0, isem1, isem2, isem3, dsem0, dsem1,
                 gsem0, gsem1, gsem2):
    c = lax.axis_index("c")
    s = lax.axis_index("s")
    rows = (rows0, rows1, rows2)
    sv = (sv0, sv1, sv2, sv3)
    dv = (dv0, dv1)
    isem = (isem0, isem1, isem2, isem3)
    dsem = (dsem0, dsem1)
    gsem = (gsem0, gsem1, gsem2)

    base = (c * NS + s) * EPW

    def issue_isrc(chunk, slot):
        pltpu.async_copy(eidx.at[pl.ds(base + chunk * CHUNK, CHUNK)],
                         sv[slot], isem[slot])

    def wait_isrc(slot):
        pltpu.make_async_copy(eidx.at[pl.ds(0, CHUNK)], sv[slot],
                              isem[slot]).wait()

    def issue_idst(chunk, slot):
        pltpu.async_copy(eidx.at[pl.ds(E + base + chunk * CHUNK, CHUNK)],
                         dv[slot], dsem[slot])

    def wait_idst(slot):
        pltpu.make_async_copy(eidx.at[pl.ds(0, CHUNK)], dv[slot],
                              dsem[slot]).wait()

    def issue_gather(sslot, rslot):
        pltpu.async_copy(vals.at[sv[sslot]], rows[rslot], gsem[rslot])

    def wait_g(rslot):
        pltpu.make_async_copy(vals.at[pl.ds(0, CHUNK)], rows[rslot],
                              gsem[rslot]).wait()

    def scatter_sync(rslot, dslot):
        pltpu.sync_copy(rows[rslot], agg.at[dv[dslot]], add=True)

    for k in range(NSV):
        issue_isrc(k, k)
    issue_idst(0, 0)

    zvec = jnp.zeros((LANES,), jnp.float32)

    def _zb_loop(t, _):
        rows0[t // (D // LANES), pl.ds((t % (D // LANES)) * LANES, LANES)] = (
            zvec)
        return 0

    lax.fori_loop(0, ZROWS * (D // LANES), _zb_loop, 0)

    def _zero_loop(j, _):
        pltpu.async_copy(rows0.at[pl.ds(0, ZROWS)],
                         agg.at[pl.ds(s * ROWS_PW + j * ZROWS, ZROWS)],
                         gsem0)
        return 0

    lax.fori_loop(0, (ROWS_PW // ZROWS), _zero_loop, 0)
    pltpu.async_copy(rows0.at[pl.ds(0, ROWS_PW % ZROWS)],
                     agg.at[pl.ds(s * ROWS_PW + (ROWS_PW // ZROWS) * ZROWS,
                                  ROWS_PW % ZROWS)],
                     gsem0)

    def _zero_drain(j, _):
        pltpu.make_async_copy(rows0.at[pl.ds(0, ZROWS)],
                              agg.at[pl.ds(0, ZROWS)], gsem0).wait()
        return 0

    lax.fori_loop(0, (ROWS_PW // ZROWS), _zero_drain, 0)
    pltpu.make_async_copy(rows0.at[pl.ds(0, ROWS_PW % ZROWS)],
                          agg.at[pl.ds(0, ROWS_PW % ZROWS)], gsem0).wait()

    plsc.subcore_barrier()

    wait_isrc(0)
    issue_gather(0, 0)
    wait_isrc(1)
    issue_gather(1, 1)

    def _step(i, u, g4=True, g2=True, g1=True):
        wait_g(u % NR)
        if g4:
            issue_isrc(i + 4, u % NSV)
        if g2:
            wait_isrc((u + 2) % NSV)
            issue_gather((u + 2) % NSV, (u + 2) % NR)
        wait_idst(u % NDV)
        scatter_sync(u % NR, u % NDV)
        if g1:
            issue_idst(i + 1, (u + 1) % NDV)

    def _pipe_body(i2, _):
        for u in range(UNROLL):
            _step(UNROLL * i2 + u, u)
        return 0

    lax.fori_loop(0, NMAIN // UNROLL, _pipe_body, 0)

    for i in range(NMAIN, NCHUNK):
        _step(i, i, g4=i + 4 < NCHUNK, g2=i + 2 < NCHUNK,
              g1=i + 1 < NCHUNK)

    pltpu.async_copy(eidx.at[pl.ds(E + base + NCHUNK * CHUNK, TAIL)],
                     dvt, dsem0)
    pltpu.async_copy(eidx.at[pl.ds(base + NCHUNK * CHUNK, TAIL)],
                     sv0.at[pl.ds(0, TAIL)], isem0)
    pltpu.make_async_copy(eidx.at[pl.ds(0, TAIL)], dvt, dsem0).wait()
    pltpu.make_async_copy(eidx.at[pl.ds(0, TAIL)], sv0.at[pl.ds(0, TAIL)],
                          isem0).wait()
    pltpu.async_copy(vals.at[sv0.at[pl.ds(0, TAIL)]],
                     rows0.at[pl.ds(0, TAIL)], gsem0).wait()
    pltpu.sync_copy(rows0.at[pl.ds(0, TAIL)], agg.at[dvt], add=True)

    plsc.subcore_barrier()

    pltpu.sync_copy(
        agg.at[pl.ds(s * ROWS_PW, ROWS_PW)],
        out.at[c].at[pl.ds(s * ROWS_PW, ROWS_PW)],
    )


def _segsum_sc(vals, edge_index):
    mesh = plsc.VectorSubcoreMesh(
        core_axis_name="c", subcore_axis_name="s",
        num_cores=NC, num_subcores=NS,
    )
    f = pl.kernel(
        _segsum_body,
        out_type=jax.ShapeDtypeStruct((NC, NPAD, D), jnp.float32),
        mesh=mesh,
        scratch_types=(
            [pltpu.VMEM((CHUNK,), jnp.int32)] * NSV
            + [pltpu.VMEM((CHUNK,), jnp.int32)] * NDV
            + [pltpu.VMEM((TAIL,), jnp.int32)]
            + [pltpu.VMEM((CHUNK, D), jnp.float32)] * NR
            + [pltpu.VMEM_SHARED((NPAD, D), jnp.float32)]
            + [pltpu.SemaphoreType.DMA] * (NSV + NDV + NR)
        ),
    )
    return f(vals, edge_index.reshape(2 * E))


_BM = 2000
_ROW_SPEC = pl.BlockSpec((_BM, D), lambda i: (i, 0))
_W_SPEC = pl.BlockSpec((D, D), lambda i: (0, 0))


def _root_body(xr, wroot, br, o):
    o[...] = jnp.dot(xr[...], wroot[...],
                     preferred_element_type=jnp.float32) + br[...]


def _root_tc(xin, wroot, b):
    return pl.pallas_call(
        _root_body,
        grid=(N // _BM,),
        in_specs=[_ROW_SPEC, _W_SPEC, pl.BlockSpec((1, D), lambda i: (0, 0))],
        out_specs=_ROW_SPEC,
        out_shape=jax.ShapeDtypeStruct((N, D), jnp.float32),
    )(xin, wroot, b.reshape(1, D))


def _combine_body(relu, a0, a1, root, wrel, o):
    acc = jnp.dot(a0[0] + a1[0], wrel[...],
                  preferred_element_type=jnp.float32)
    acc += root[...]
    o[...] = jnp.maximum(acc, 0.0) if relu else acc


def _combine_tc(aggp, root, wrel, relu):
    return pl.pallas_call(
        functools.partial(_combine_body, relu),
        grid=(N // _BM,),
        in_specs=[pl.BlockSpec((1, _BM, D), lambda i: (0, i, 0)),
                  pl.BlockSpec((1, _BM, D), lambda i: (1, i, 0)),
                  _ROW_SPEC, _W_SPEC],
        out_specs=_ROW_SPEC,
        out_shape=jax.ShapeDtypeStruct((N, D), jnp.float32),
    )(aggp, aggp, root, wrel)


def kernel(x, edge_index, W_rel0, b_rel0, W_root0, W_rel1, b_rel1, W_root1):
    root0 = _root_tc(x, W_root0, b_rel0)
    agg0 = _segsum_sc(x, edge_index)
    h = _combine_tc(agg0, root0, W_rel0, relu=True)
    root1 = _root_tc(h, W_root1, b_rel1)
    agg1 = _segsum_sc(h, edge_index)
    out = _combine_tc(agg1, root1, W_rel1, relu=False)
    return out
